# baseline (device time: 17756 ns/iter reference)
import jax
import jax.numpy as jnp
from jax import lax
from jax.experimental import pallas as pl
from jax.experimental.pallas import tpu as pltpu

S = 8


def kernel(x, dy):
    k_per, d = x.shape
    _, f = dy.shape
    out_rows = d // 2
    half = out_rows // 2
    sub = half // S

    contract = (((0,), (0,)), ((), ()))

    def body(x_ref, dy_ref, out_ref, acc_ref, send_ref, recv1_ref, recv2_ref,
             sems1_send, sems1_recv, sems2_send, sems2_recv):
        my_x = lax.axis_index("x")
        my_y = lax.axis_index("y")
        x_nbr = (1 - my_x, my_y)
        y_nbr = (my_x, 1 - my_y)

        barrier_sem = pltpu.get_barrier_semaphore()
        for nbr in (x_nbr, y_nbr):
            pl.semaphore_signal(
                barrier_sem, inc=1,
                device_id=nbr, device_id_type=pl.DeviceIdType.MESH,
            )
        pl.semaphore_wait(barrier_sem, 2)

        dyb = dy_ref[...].astype(jnp.bfloat16)

        send_off = (1 - my_x) * out_rows + my_y * half
        xs = x_ref[:, pl.ds(send_off, half)].astype(jnp.bfloat16)
        send_ref[...] = lax.dot_general(
            xs, dyb, contract, preferred_element_type=jnp.float32,
        ).astype(jnp.bfloat16)
        rdma1 = []
        for s in range(S):
            r = pltpu.make_async_remote_copy(
                src_ref=send_ref.at[pl.ds(s * sub, sub)],
                dst_ref=recv1_ref.at[pl.ds(s * sub, sub)],
                send_sem=sems1_send.at[s], recv_sem=sems1_recv.at[s],
                device_id=x_nbr, device_id_type=pl.DeviceIdType.MESH,
            )
            r.start()
            rdma1.append(r)

        base = my_x * out_rows
        xo = x_ref[:, pl.ds(base, out_rows)].astype(jnp.bfloat16)
        acc_ref[...] = lax.dot_general(
            xo, dyb, contract, preferred_element_type=jnp.float32,
        )

        off1 = my_y * half
        off2 = (1 - my_y) * half
        rdma2 = []
        for s in range(S):
            rdma1[s].wait_recv()
            r = pltpu.make_async_remote_copy(
                src_ref=recv1_ref.at[pl.ds(s * sub, sub)],
                dst_ref=recv2_ref.at[pl.ds(s * sub, sub)],
                send_sem=sems2_send.at[s], recv_sem=sems2_recv.at[s],
                device_id=y_nbr, device_id_type=pl.DeviceIdType.MESH,
            )
            r.start()
            rdma2.append(r)
            rows = pl.ds(off1 + s * sub, sub)
            out_ref[rows, :] = (
                acc_ref[rows, :]
                + recv1_ref[pl.ds(s * sub, sub), :].astype(jnp.float32)
            ).astype(jnp.bfloat16)

        for s in range(S):
            rdma2[s].wait_recv()
            rows = pl.ds(off2 + s * sub, sub)
            out_ref[rows, :] = (
                acc_ref[rows, :]
                + recv2_ref[pl.ds(s * sub, sub), :].astype(jnp.float32)
            ).astype(jnp.bfloat16)

        for s in range(S):
            rdma1[s].wait_send()
            rdma2[s].wait_send()

    return pl.pallas_call(
        body,
        out_shape=jax.ShapeDtypeStruct((out_rows, f), jnp.bfloat16),
        in_specs=[
            pl.BlockSpec(memory_space=pltpu.VMEM),
            pl.BlockSpec(memory_space=pltpu.VMEM),
        ],
        out_specs=pl.BlockSpec(memory_space=pltpu.VMEM),
        scratch_shapes=[
            pltpu.VMEM((out_rows, f), jnp.float32),
            pltpu.VMEM((half, f), jnp.bfloat16),
            pltpu.VMEM((half, f), jnp.bfloat16),
            pltpu.VMEM((half, f), jnp.bfloat16),
            pltpu.SemaphoreType.DMA((S,)),
            pltpu.SemaphoreType.DMA((S,)),
            pltpu.SemaphoreType.DMA((S,)),
            pltpu.SemaphoreType.DMA((S,)),
        ],
        compiler_params=pltpu.CompilerParams(collective_id=0),
    )(x, dy)


# device time: 17755 ns/iter; 1.0001x vs baseline; 1.0001x over previous
import jax
import jax.numpy as jnp
from jax import lax
from jax.experimental import pallas as pl
from jax.experimental.pallas import tpu as pltpu

C = 4


def kernel(x, dy):
    k_per, d = x.shape
    _, f = dy.shape
    out_rows = d // 2
    half = out_rows // 2
    cw = f // C

    contract = (((0,), (0,)), ((), ()))

    def body(x_ref, dy_ref, out_ref, dyb_ref, acc_ref, send_ref, recv1_ref,
             recv2_ref, sems1_send, sems1_recv, sems2_send, sems2_recv):
        my_x = lax.axis_index("x")
        my_y = lax.axis_index("y")
        x_nbr = (1 - my_x, my_y)
        y_nbr = (my_x, 1 - my_y)

        barrier_sem = pltpu.get_barrier_semaphore()
        for nbr in (x_nbr, y_nbr):
            pl.semaphore_signal(
                barrier_sem, inc=1,
                device_id=nbr, device_id_type=pl.DeviceIdType.MESH,
            )

        send_off = (1 - my_x) * out_rows + my_y * half
        xs = x_ref[:, pl.ds(send_off, half)].astype(jnp.bfloat16)
        cols0 = pl.ds(0, cw)
        dyb_ref[:, cols0] = dy_ref[:, cols0].astype(jnp.bfloat16)
        send_ref[0] = lax.dot_general(
            xs, dyb_ref[:, cols0], contract,
            preferred_element_type=jnp.float32,
        ).astype(jnp.bfloat16)

        pl.semaphore_wait(barrier_sem, 2)

        rdma1 = []
        for c in range(C):
            if c > 0:
                cols = pl.ds(c * cw, cw)
                dyb_ref[:, cols] = dy_ref[:, cols].astype(jnp.bfloat16)
                send_ref[c] = lax.dot_general(
                    xs, dyb_ref[:, cols], contract,
                    preferred_element_type=jnp.float32,
                ).astype(jnp.bfloat16)
            r = pltpu.make_async_remote_copy(
                src_ref=send_ref.at[c], dst_ref=recv1_ref.at[c],
                send_sem=sems1_send.at[c], recv_sem=sems1_recv.at[c],
                device_id=x_nbr, device_id_type=pl.DeviceIdType.MESH,
            )
            r.start()
            rdma1.append(r)

        base = my_x * out_rows
        xo = x_ref[:, pl.ds(base, out_rows)].astype(jnp.bfloat16)
        acc_ref[...] = lax.dot_general(
            xo, dyb_ref[...], contract, preferred_element_type=jnp.float32,
        )

        off1 = my_y * half
        off2 = (1 - my_y) * half
        rows1 = pl.ds(off1, half)
        rows2 = pl.ds(off2, half)
        rdma2 = []
        for c in range(C):
            rdma1[c].wait_recv()
            r = pltpu.make_async_remote_copy(
                src_ref=recv1_ref.at[c], dst_ref=recv2_ref.at[c],
                send_sem=sems2_send.at[c], recv_sem=sems2_recv.at[c],
                device_id=y_nbr, device_id_type=pl.DeviceIdType.MESH,
            )
            r.start()
            rdma2.append(r)
            cols = pl.ds(c * cw, cw)
            out_ref[rows1, cols] = (
                acc_ref[rows1, cols] + recv1_ref[c].astype(jnp.float32)
            ).astype(jnp.bfloat16)

        for c in range(C):
            rdma2[c].wait_recv()
            cols = pl.ds(c * cw, cw)
            out_ref[rows2, cols] = (
                acc_ref[rows2, cols] + recv2_ref[c].astype(jnp.float32)
            ).astype(jnp.bfloat16)

        for c in range(C):
            rdma1[c].wait_send()
            rdma2[c].wait_send()

    return pl.pallas_call(
        body,
        out_shape=jax.ShapeDtypeStruct((out_rows, f), jnp.bfloat16),
        in_specs=[
            pl.BlockSpec(memory_space=pltpu.VMEM),
            pl.BlockSpec(memory_space=pltpu.VMEM),
        ],
        out_specs=pl.BlockSpec(memory_space=pltpu.VMEM),
        scratch_shapes=[
            pltpu.VMEM((k_per, f), jnp.bfloat16),
            pltpu.VMEM((out_rows, f), jnp.float32),
            pltpu.VMEM((C, half, cw), jnp.bfloat16),
            pltpu.VMEM((C, half, cw), jnp.bfloat16),
            pltpu.VMEM((C, half, cw), jnp.bfloat16),
            pltpu.SemaphoreType.DMA((C,)),
            pltpu.SemaphoreType.DMA((C,)),
            pltpu.SemaphoreType.DMA((C,)),
            pltpu.SemaphoreType.DMA((C,)),
        ],
        compiler_params=pltpu.CompilerParams(collective_id=0),
    )(x, dy)


# device time: 17106 ns/iter; 1.0380x vs baseline; 1.0379x over previous
import jax
import jax.numpy as jnp
from jax import lax
from jax.experimental import pallas as pl
from jax.experimental.pallas import tpu as pltpu

C = 8


def kernel(x, dy):
    k_per, d = x.shape
    _, f = dy.shape
    out_rows = d // 2
    half = out_rows // 2
    cw = f // C

    contract = (((0,), (0,)), ((), ()))

    def body(x_ref, dy_ref, out_ref, dyb_ref, acc_ref, send_ref, recv1_ref,
             recv2_ref, sems1_send, sems1_recv, sems2_send, sems2_recv):
        my_x = lax.axis_index("x")
        my_y = lax.axis_index("y")
        x_nbr = (1 - my_x, my_y)
        y_nbr = (my_x, 1 - my_y)

        barrier_sem = pltpu.get_barrier_semaphore()
        for nbr in (x_nbr, y_nbr):
            pl.semaphore_signal(
                barrier_sem, inc=1,
                device_id=nbr, device_id_type=pl.DeviceIdType.MESH,
            )

        send_off = (1 - my_x) * out_rows + my_y * half
        xs = x_ref[:, pl.ds(send_off, half)].astype(jnp.bfloat16)
        cols0 = pl.ds(0, cw)
        dyb_ref[:, cols0] = dy_ref[:, cols0].astype(jnp.bfloat16)
        send_ref[0] = lax.dot_general(
            xs, dyb_ref[:, cols0], contract,
            preferred_element_type=jnp.float32,
        ).astype(jnp.bfloat16)

        pl.semaphore_wait(barrier_sem, 2)

        rdma1 = []
        for c in range(C):
            if c > 0:
                cols = pl.ds(c * cw, cw)
                dyb_ref[:, cols] = dy_ref[:, cols].astype(jnp.bfloat16)
                send_ref[c] = lax.dot_general(
                    xs, dyb_ref[:, cols], contract,
                    preferred_element_type=jnp.float32,
                ).astype(jnp.bfloat16)
            r = pltpu.make_async_remote_copy(
                src_ref=send_ref.at[c], dst_ref=recv1_ref.at[c],
                send_sem=sems1_send.at[c], recv_sem=sems1_recv.at[c],
                device_id=x_nbr, device_id_type=pl.DeviceIdType.MESH,
            )
            r.start()
            rdma1.append(r)

        base = my_x * out_rows
        xo = x_ref[:, pl.ds(base, out_rows)].astype(jnp.bfloat16)
        acc_ref[...] = lax.dot_general(
            xo, dyb_ref[...], contract, preferred_element_type=jnp.float32,
        )

        off1 = my_y * half
        off2 = (1 - my_y) * half
        rows1 = pl.ds(off1, half)
        rows2 = pl.ds(off2, half)
        rdma2 = []
        for c in range(C):
            rdma1[c].wait_recv()
            r = pltpu.make_async_remote_copy(
                src_ref=recv1_ref.at[c], dst_ref=recv2_ref.at[c],
                send_sem=sems2_send.at[c], recv_sem=sems2_recv.at[c],
                device_id=y_nbr, device_id_type=pl.DeviceIdType.MESH,
            )
            r.start()
            rdma2.append(r)
            cols = pl.ds(c * cw, cw)
            out_ref[rows1, cols] = (
                acc_ref[rows1, cols] + recv1_ref[c].astype(jnp.float32)
            ).astype(jnp.bfloat16)

        for c in range(C):
            rdma2[c].wait_recv()
            cols = pl.ds(c * cw, cw)
            out_ref[rows2, cols] = (
                acc_ref[rows2, cols] + recv2_ref[c].astype(jnp.float32)
            ).astype(jnp.bfloat16)

        for c in range(C):
            rdma1[c].wait_send()
            rdma2[c].wait_send()

    return pl.pallas_call(
        body,
        out_shape=jax.ShapeDtypeStruct((out_rows, f), jnp.bfloat16),
        in_specs=[
            pl.BlockSpec(memory_space=pltpu.VMEM),
            pl.BlockSpec(memory_space=pltpu.VMEM),
        ],
        out_specs=pl.BlockSpec(memory_space=pltpu.VMEM),
        scratch_shapes=[
            pltpu.VMEM((k_per, f), jnp.bfloat16),
            pltpu.VMEM((out_rows, f), jnp.float32),
            pltpu.VMEM((C, half, cw), jnp.bfloat16),
            pltpu.VMEM((C, half, cw), jnp.bfloat16),
            pltpu.VMEM((C, half, cw), jnp.bfloat16),
            pltpu.SemaphoreType.DMA((C,)),
            pltpu.SemaphoreType.DMA((C,)),
            pltpu.SemaphoreType.DMA((C,)),
            pltpu.SemaphoreType.DMA((C,)),
        ],
        compiler_params=pltpu.CompilerParams(collective_id=0),
    )(x, dy)
